# baseline (speedup 1.0000x reference)
import numpy as np
import jax
import jax.numpy as jnp
from jax.experimental import pallas as pl
from jax.experimental.pallas import tpu as pltpu

_FEAT_STRIDE = 16.0
_TOPN = 300
_A = 9
_H = 64
_W = 64
_LANES = 128
_ROWS = (_H * _W * _A) // _LANES
_N = _ROWS * _LANES


def _anchor_table():
    base_size = 16
    ratios = np.array([0.5, 1.0, 2.0], dtype=np.float64)
    scales = np.array([8.0, 16.0, 32.0], dtype=np.float64)
    x_ctr = 0.5 * (base_size - 1)
    y_ctr = 0.5 * (base_size - 1)
    size = float(base_size * base_size)
    ws = np.round(np.sqrt(size / ratios))
    hs = np.round(ws * ratios)

    def mk(ws_, hs_, xc, yc):
        ws_ = np.asarray(ws_, dtype=np.float64).reshape(-1, 1)
        hs_ = np.asarray(hs_, dtype=np.float64).reshape(-1, 1)
        return np.hstack((xc - 0.5 * (ws_ - 1), yc - 0.5 * (hs_ - 1),
                          xc + 0.5 * (ws_ - 1), yc + 0.5 * (hs_ - 1)))

    ratio_anchors = mk(ws, hs, x_ctr, y_ctr)
    rows = []
    for a in ratio_anchors:
        w = a[2] - a[0] + 1.0
        h = a[3] - a[1] + 1.0
        xc = a[0] + 0.5 * (w - 1)
        yc = a[1] + 0.5 * (h - 1)
        rows.append(mk(w * scales, h * scales, xc, yc))
    return np.vstack(rows).astype(np.float32)


_TAB = _anchor_table()
_AW = (_TAB[:, 2] - _TAB[:, 0] + 1.0).tolist()
_AH = (_TAB[:, 3] - _TAB[:, 1] + 1.0).tolist()
_ACX = (_TAB[:, 0] + 0.5 * (_TAB[:, 2] - _TAB[:, 0] + 1.0)).tolist()
_ACY = (_TAB[:, 1] + 0.5 * (_TAB[:, 3] - _TAB[:, 1] + 1.0)).tolist()


def _prop_kernel(im_ref, td_ref, sc_ref, scT_ref, dx_ref, dy_ref, dw_ref, dh_ref,
                 out_ref, keys_ref, selv_ref, seli_ref):
    b = pl.program_id(0)
    keys_ref[:] = sc_ref[0]
    lane = jax.lax.broadcasted_iota(jnp.int32, (1, _LANES), 1)
    laneR = jax.lax.broadcasted_iota(jnp.int32, (1, _ROWS), 1)
    colmax0 = jnp.max(scT_ref[0], axis=0, keepdims=True)

    def body(it, colmax):
        m = jnp.max(colmax)
        r = jnp.min(jnp.where(colmax == m, laneR, _ROWS))
        row = keys_ref[pl.ds(r, 1), :]
        c = jnp.min(jnp.where(row == m, lane, _LANES))
        e = r * _LANES + c
        onehot = lane == c

        def pick(ref):
            prow = ref[0, pl.ds(r, 1), :]
            return jnp.sum(jnp.where(onehot, prow, 0.0))

        dxv = pick(dx_ref)
        dyv = pick(dy_ref)
        dwv = pick(dw_ref)
        dhv = pick(dh_ref)
        valrow = (jnp.where(lane == 0, m, 0.0)
                  + jnp.where(lane == 1, dxv, 0.0)
                  + jnp.where(lane == 2, dyv, 0.0)
                  + jnp.where(lane == 3, dwv, 0.0)
                  + jnp.where(lane == 4, dhv, 0.0))
        selv_ref[pl.ds(it, 1), :] = valrow
        seli_ref[pl.ds(it, 1), :] = jnp.broadcast_to(e, (1, _LANES))
        row2 = jnp.where(onehot, -jnp.inf, row)
        keys_ref[pl.ds(r, 1), :] = row2
        return jnp.where(laneR == r, jnp.max(row2), colmax)

    jax.lax.fori_loop(0, _TOPN, body, colmax0, unroll=False)

    sv = selv_ref[:]
    score = sv[:, 0:1]
    dx = sv[:, 1:2]
    dy = sv[:, 2:3]
    dw = sv[:, 3:4]
    dh = sv[:, 4:5]
    ei = seli_ref[:, 0:1]

    a = ei % _A
    cell = ei // _A
    wi = cell % _W
    hi = cell // _W
    sx = wi.astype(jnp.float32) * _FEAT_STRIDE
    sy = hi.astype(jnp.float32) * _FEAT_STRIDE

    zero = jnp.zeros_like(score)
    aw = zero
    ah = zero
    acx = zero
    acy = zero
    for k in range(_A):
        sel = a == k
        aw = jnp.where(sel, _AW[k], aw)
        ah = jnp.where(sel, _AH[k], ah)
        acx = jnp.where(sel, _ACX[k], acx)
        acy = jnp.where(sel, _ACY[k], acy)
    ctr_x = acx + sx
    ctr_y = acy + sy
    pcx = dx * aw + ctr_x
    pcy = dy * ah + ctr_y
    pw = jnp.exp(dw) * aw
    ph = jnp.exp(dh) * ah
    px1 = pcx - 0.5 * pw
    py1 = pcy - 0.5 * ph
    px2 = pcx + 0.5 * pw
    py2 = pcy + 0.5 * ph

    imrow = im_ref[pl.ds(b, 1), :]
    ym = imrow[0, 0] - 1.0
    xm = imrow[0, 1] - 1.0
    px1 = jnp.clip(px1, 0.0, xm)
    py1 = jnp.clip(py1, 0.0, ym)
    px2 = jnp.clip(px2, 0.0, xm)
    py2 = jnp.clip(py2, 0.0, ym)

    z2 = td_ref[0, 0] - 1.0
    bcol = b.astype(jnp.float32)
    li = jax.lax.broadcasted_iota(jnp.int32, (_TOPN, 8), 1)
    out = jnp.where(li == 0, bcol, 0.0)
    out = jnp.where(li == 1, px1, out)
    out = jnp.where(li == 2, py1, out)
    out = jnp.where(li == 4, px2, out)
    out = jnp.where(li == 5, py2, out)
    out = jnp.where(li == 6, z2, out)
    out = jnp.where(li == 7, score, out)
    out_ref[0] = out


def kernel(scores, bbox_frame, im_info, time_dim):
    B = scores.shape[0]
    sc = jnp.transpose(scores[:, _A:2 * _A], (0, 2, 3, 1)).reshape(B, _ROWS, _LANES)
    scT = jnp.transpose(sc, (0, 2, 1))
    dxp = jnp.transpose(bbox_frame[:, 0::4], (0, 2, 3, 1)).reshape(B, _ROWS, _LANES)
    dyp = jnp.transpose(bbox_frame[:, 1::4], (0, 2, 3, 1)).reshape(B, _ROWS, _LANES)
    dwp = jnp.transpose(bbox_frame[:, 2::4], (0, 2, 3, 1)).reshape(B, _ROWS, _LANES)
    dhp = jnp.transpose(bbox_frame[:, 3::4], (0, 2, 3, 1)).reshape(B, _ROWS, _LANES)
    td = jnp.asarray(time_dim, jnp.float32).reshape(1, 1)

    plane = pl.BlockSpec((1, _ROWS, _LANES), lambda b: (b, 0, 0))
    out = pl.pallas_call(
        _prop_kernel,
        grid=(B,),
        in_specs=[
            pl.BlockSpec((B, 3), lambda b: (0, 0)),
            pl.BlockSpec((1, 1), lambda b: (0, 0)),
            plane,
            pl.BlockSpec((1, _LANES, _ROWS), lambda b: (b, 0, 0)),
            plane, plane, plane, plane,
        ],
        out_specs=pl.BlockSpec((1, _TOPN, 8), lambda b: (b, 0, 0)),
        out_shape=jax.ShapeDtypeStruct((B, _TOPN, 8), jnp.float32),
        scratch_shapes=[
            pltpu.VMEM((_ROWS, _LANES), jnp.float32),
            pltpu.VMEM((_TOPN, _LANES), jnp.float32),
            pltpu.VMEM((_TOPN, _LANES), jnp.int32),
        ],
    )(im_info, td, sc, scT, dxp, dyp, dwp, dhp)
    return out


# speedup vs baseline: 1.5040x; 1.5040x over previous
import numpy as np
import jax
import jax.numpy as jnp
from jax.experimental import pallas as pl
from jax.experimental.pallas import tpu as pltpu

_FEAT_STRIDE = 16.0
_TOPN = 300
_A = 9
_H = 64
_W = 64
_LANES = 128
_ROWS = (_H * _W * _A) // _LANES
_N = _ROWS * _LANES
_B = 4


def _anchor_table():
    base_size = 16
    ratios = np.array([0.5, 1.0, 2.0], dtype=np.float64)
    scales = np.array([8.0, 16.0, 32.0], dtype=np.float64)
    x_ctr = 0.5 * (base_size - 1)
    y_ctr = 0.5 * (base_size - 1)
    size = float(base_size * base_size)
    ws = np.round(np.sqrt(size / ratios))
    hs = np.round(ws * ratios)

    def mk(ws_, hs_, xc, yc):
        ws_ = np.asarray(ws_, dtype=np.float64).reshape(-1, 1)
        hs_ = np.asarray(hs_, dtype=np.float64).reshape(-1, 1)
        return np.hstack((xc - 0.5 * (ws_ - 1), yc - 0.5 * (hs_ - 1),
                          xc + 0.5 * (ws_ - 1), yc + 0.5 * (hs_ - 1)))

    ratio_anchors = mk(ws, hs, x_ctr, y_ctr)
    rows = []
    for a in ratio_anchors:
        w = a[2] - a[0] + 1.0
        h = a[3] - a[1] + 1.0
        xc = a[0] + 0.5 * (w - 1)
        yc = a[1] + 0.5 * (h - 1)
        rows.append(mk(w * scales, h * scales, xc, yc))
    return np.vstack(rows).astype(np.float32)


_TAB = _anchor_table()
_AW = (_TAB[:, 2] - _TAB[:, 0] + 1.0).tolist()
_AH = (_TAB[:, 3] - _TAB[:, 1] + 1.0).tolist()
_ACX = (_TAB[:, 0] + 0.5 * (_TAB[:, 2] - _TAB[:, 0] + 1.0)).tolist()
_ACY = (_TAB[:, 1] + 0.5 * (_TAB[:, 3] - _TAB[:, 1] + 1.0)).tolist()


def _prop_kernel(im_ref, td_ref, sc_ref, dx_ref, dy_ref, dw_ref, dh_ref,
                 out_ref, selv_ref, seli_ref):
    iota_r = jax.lax.broadcasted_iota(jnp.int32, (_ROWS, _LANES), 0)
    iota_c = jax.lax.broadcasted_iota(jnp.int32, (_ROWS, _LANES), 1)
    flat = iota_r * _LANES + iota_c
    lane = jax.lax.broadcasted_iota(jnp.int32, (1, _LANES), 1)

    def body(it, keys):
        new_keys = []
        for b in range(_B):
            kb = keys[b]
            m = jnp.max(kb)
            e = jnp.min(jnp.where(kb == m, flat, _N))
            r = e // _LANES
            c = e - r * _LANES
            onehot = lane == c

            def pick(ref):
                prow = ref[b, pl.ds(r, 1), :]
                return jnp.sum(jnp.where(onehot, prow, 0.0))

            dxv = pick(dx_ref)
            dyv = pick(dy_ref)
            dwv = pick(dw_ref)
            dhv = pick(dh_ref)
            valrow = (jnp.where(lane == 0, m, 0.0)
                      + jnp.where(lane == 1, dxv, 0.0)
                      + jnp.where(lane == 2, dyv, 0.0)
                      + jnp.where(lane == 3, dwv, 0.0)
                      + jnp.where(lane == 4, dhv, 0.0))
            selv_ref[b, pl.ds(it, 1), :] = valrow
            seli_ref[b, pl.ds(it, 1), :] = jnp.broadcast_to(e, (1, _LANES))
            new_keys.append(jnp.where(flat == e, -jnp.inf, kb))
        return tuple(new_keys)

    keys0 = tuple(sc_ref[b] for b in range(_B))
    jax.lax.fori_loop(0, _TOPN, body, keys0, unroll=False)

    z2 = td_ref[0, 0] - 1.0
    for b in range(_B):
        sv = selv_ref[b]
        score = sv[:, 0:1]
        dx = sv[:, 1:2]
        dy = sv[:, 2:3]
        dw = sv[:, 3:4]
        dh = sv[:, 4:5]
        ei = seli_ref[b, :, 0:1]

        a = ei % _A
        cell = ei // _A
        wi = cell % _W
        hi = cell // _W
        sx = wi.astype(jnp.float32) * _FEAT_STRIDE
        sy = hi.astype(jnp.float32) * _FEAT_STRIDE

        zero = jnp.zeros_like(score)
        aw = zero
        ah = zero
        acx = zero
        acy = zero
        for k in range(_A):
            sel = a == k
            aw = jnp.where(sel, _AW[k], aw)
            ah = jnp.where(sel, _AH[k], ah)
            acx = jnp.where(sel, _ACX[k], acx)
            acy = jnp.where(sel, _ACY[k], acy)
        ctr_x = acx + sx
        ctr_y = acy + sy
        pcx = dx * aw + ctr_x
        pcy = dy * ah + ctr_y
        pw = jnp.exp(dw) * aw
        ph = jnp.exp(dh) * ah
        px1 = pcx - 0.5 * pw
        py1 = pcy - 0.5 * ph
        px2 = pcx + 0.5 * pw
        py2 = pcy + 0.5 * ph

        imrow = im_ref[pl.ds(b, 1), :]
        ym = imrow[0, 0] - 1.0
        xm = imrow[0, 1] - 1.0
        px1 = jnp.clip(px1, 0.0, xm)
        py1 = jnp.clip(py1, 0.0, ym)
        px2 = jnp.clip(px2, 0.0, xm)
        py2 = jnp.clip(py2, 0.0, ym)

        li = jax.lax.broadcasted_iota(jnp.int32, (_TOPN, 8), 1)
        out = jnp.where(li == 0, float(b), 0.0)
        out = jnp.where(li == 1, px1, out)
        out = jnp.where(li == 2, py1, out)
        out = jnp.where(li == 4, px2, out)
        out = jnp.where(li == 5, py2, out)
        out = jnp.where(li == 6, z2, out)
        out = jnp.where(li == 7, score, out)
        out_ref[b] = out


def kernel(scores, bbox_frame, im_info, time_dim):
    B = scores.shape[0]
    sc = jnp.transpose(scores[:, _A:2 * _A], (0, 2, 3, 1)).reshape(B, _ROWS, _LANES)
    dxp = jnp.transpose(bbox_frame[:, 0::4], (0, 2, 3, 1)).reshape(B, _ROWS, _LANES)
    dyp = jnp.transpose(bbox_frame[:, 1::4], (0, 2, 3, 1)).reshape(B, _ROWS, _LANES)
    dwp = jnp.transpose(bbox_frame[:, 2::4], (0, 2, 3, 1)).reshape(B, _ROWS, _LANES)
    dhp = jnp.transpose(bbox_frame[:, 3::4], (0, 2, 3, 1)).reshape(B, _ROWS, _LANES)
    td = jnp.asarray(time_dim, jnp.float32).reshape(1, 1)

    plane = pl.BlockSpec((B, _ROWS, _LANES), lambda: (0, 0, 0))
    out = pl.pallas_call(
        _prop_kernel,
        grid=(),
        in_specs=[
            pl.BlockSpec((B, 3), lambda: (0, 0)),
            pl.BlockSpec((1, 1), lambda: (0, 0)),
            plane, plane, plane, plane, plane,
        ],
        out_specs=pl.BlockSpec((B, _TOPN, 8), lambda: (0, 0, 0)),
        out_shape=jax.ShapeDtypeStruct((B, _TOPN, 8), jnp.float32),
        scratch_shapes=[
            pltpu.VMEM((_B, _TOPN, _LANES), jnp.float32),
            pltpu.VMEM((_B, _TOPN, _LANES), jnp.int32),
        ],
    )(im_info, td, sc, dxp, dyp, dwp, dhp)
    return out


# speedup vs baseline: 1.6389x; 1.0896x over previous
import numpy as np
import jax
import jax.numpy as jnp
from jax.experimental import pallas as pl
from jax.experimental.pallas import tpu as pltpu

_FEAT_STRIDE = 16.0
_TOPN = 300
_A = 9
_H = 64
_W = 64
_LANES = 128
_ROWS = (_H * _W * _A) // _LANES
_N = _ROWS * _LANES
_B = 4


def _anchor_table():
    base_size = 16
    ratios = np.array([0.5, 1.0, 2.0], dtype=np.float64)
    scales = np.array([8.0, 16.0, 32.0], dtype=np.float64)
    x_ctr = 0.5 * (base_size - 1)
    y_ctr = 0.5 * (base_size - 1)
    size = float(base_size * base_size)
    ws = np.round(np.sqrt(size / ratios))
    hs = np.round(ws * ratios)

    def mk(ws_, hs_, xc, yc):
        ws_ = np.asarray(ws_, dtype=np.float64).reshape(-1, 1)
        hs_ = np.asarray(hs_, dtype=np.float64).reshape(-1, 1)
        return np.hstack((xc - 0.5 * (ws_ - 1), yc - 0.5 * (hs_ - 1),
                          xc + 0.5 * (ws_ - 1), yc + 0.5 * (hs_ - 1)))

    ratio_anchors = mk(ws, hs, x_ctr, y_ctr)
    rows = []
    for a in ratio_anchors:
        w = a[2] - a[0] + 1.0
        h = a[3] - a[1] + 1.0
        xc = a[0] + 0.5 * (w - 1)
        yc = a[1] + 0.5 * (h - 1)
        rows.append(mk(w * scales, h * scales, xc, yc))
    return np.vstack(rows).astype(np.float32)


_TAB = _anchor_table()
_AW = (_TAB[:, 2] - _TAB[:, 0] + 1.0).tolist()
_AH = (_TAB[:, 3] - _TAB[:, 1] + 1.0).tolist()
_ACX = (_TAB[:, 0] + 0.5 * (_TAB[:, 2] - _TAB[:, 0] + 1.0)).tolist()
_ACY = (_TAB[:, 1] + 0.5 * (_TAB[:, 3] - _TAB[:, 1] + 1.0)).tolist()


def _prop_kernel(im_ref, td_ref, sc_ref, dx_ref, dy_ref, dw_ref, dh_ref,
                 out_ref, ssc_ref, sdx_ref, sdy_ref, sdw_ref, sdh_ref, seli_ref):
    iota_r = jax.lax.broadcasted_iota(jnp.int32, (_ROWS, _LANES), 0)
    iota_c = jax.lax.broadcasted_iota(jnp.int32, (_ROWS, _LANES), 1)
    flat = iota_r * _LANES + iota_c
    lane = jax.lax.broadcasted_iota(jnp.int32, (1, _LANES), 1)

    def body(it, keys):
        new_keys = []
        for b in range(_B):
            kb = keys[b]
            e = jnp.argmax(kb).astype(jnp.int32)
            r = e // _LANES
            c = e - r * _LANES
            onehot = lane == c
            ssc_ref[b, pl.ds(it, 1), :] = jnp.where(onehot, sc_ref[b, pl.ds(r, 1), :], 0.0)
            sdx_ref[b, pl.ds(it, 1), :] = jnp.where(onehot, dx_ref[b, pl.ds(r, 1), :], 0.0)
            sdy_ref[b, pl.ds(it, 1), :] = jnp.where(onehot, dy_ref[b, pl.ds(r, 1), :], 0.0)
            sdw_ref[b, pl.ds(it, 1), :] = jnp.where(onehot, dw_ref[b, pl.ds(r, 1), :], 0.0)
            sdh_ref[b, pl.ds(it, 1), :] = jnp.where(onehot, dh_ref[b, pl.ds(r, 1), :], 0.0)
            seli_ref[b, pl.ds(it, 1), :] = jnp.broadcast_to(e, (1, _LANES))
            new_keys.append(jnp.where(flat == e, -jnp.inf, kb))
        return tuple(new_keys)

    keys0 = tuple(sc_ref[b] for b in range(_B))
    jax.lax.fori_loop(0, _TOPN, body, keys0, unroll=False)

    z2 = td_ref[0, 0] - 1.0
    for b in range(_B):
        score = jnp.sum(ssc_ref[b], axis=1, keepdims=True)
        dx = jnp.sum(sdx_ref[b], axis=1, keepdims=True)
        dy = jnp.sum(sdy_ref[b], axis=1, keepdims=True)
        dw = jnp.sum(sdw_ref[b], axis=1, keepdims=True)
        dh = jnp.sum(sdh_ref[b], axis=1, keepdims=True)
        ei = seli_ref[b, :, 0:1]

        a = ei % _A
        cell = ei // _A
        wi = cell % _W
        hi = cell // _W
        sx = wi.astype(jnp.float32) * _FEAT_STRIDE
        sy = hi.astype(jnp.float32) * _FEAT_STRIDE

        zero = jnp.zeros_like(score)
        aw = zero
        ah = zero
        acx = zero
        acy = zero
        for k in range(_A):
            sel = a == k
            aw = jnp.where(sel, _AW[k], aw)
            ah = jnp.where(sel, _AH[k], ah)
            acx = jnp.where(sel, _ACX[k], acx)
            acy = jnp.where(sel, _ACY[k], acy)
        ctr_x = acx + sx
        ctr_y = acy + sy
        pcx = dx * aw + ctr_x
        pcy = dy * ah + ctr_y
        pw = jnp.exp(dw) * aw
        ph = jnp.exp(dh) * ah
        px1 = pcx - 0.5 * pw
        py1 = pcy - 0.5 * ph
        px2 = pcx + 0.5 * pw
        py2 = pcy + 0.5 * ph

        imrow = im_ref[pl.ds(b, 1), :]
        ym = imrow[0, 0] - 1.0
        xm = imrow[0, 1] - 1.0
        px1 = jnp.clip(px1, 0.0, xm)
        py1 = jnp.clip(py1, 0.0, ym)
        px2 = jnp.clip(px2, 0.0, xm)
        py2 = jnp.clip(py2, 0.0, ym)

        li = jax.lax.broadcasted_iota(jnp.int32, (_TOPN, 8), 1)
        out = jnp.where(li == 0, float(b), 0.0)
        out = jnp.where(li == 1, px1, out)
        out = jnp.where(li == 2, py1, out)
        out = jnp.where(li == 4, px2, out)
        out = jnp.where(li == 5, py2, out)
        out = jnp.where(li == 6, z2, out)
        out = jnp.where(li == 7, score, out)
        out_ref[b] = out


def kernel(scores, bbox_frame, im_info, time_dim):
    B = scores.shape[0]
    sc = jnp.transpose(scores[:, _A:2 * _A], (0, 2, 3, 1)).reshape(B, _ROWS, _LANES)
    dxp = jnp.transpose(bbox_frame[:, 0::4], (0, 2, 3, 1)).reshape(B, _ROWS, _LANES)
    dyp = jnp.transpose(bbox_frame[:, 1::4], (0, 2, 3, 1)).reshape(B, _ROWS, _LANES)
    dwp = jnp.transpose(bbox_frame[:, 2::4], (0, 2, 3, 1)).reshape(B, _ROWS, _LANES)
    dhp = jnp.transpose(bbox_frame[:, 3::4], (0, 2, 3, 1)).reshape(B, _ROWS, _LANES)
    td = jnp.asarray(time_dim, jnp.float32).reshape(1, 1)

    plane = pl.BlockSpec((B, _ROWS, _LANES), lambda: (0, 0, 0))
    out = pl.pallas_call(
        _prop_kernel,
        grid=(),
        in_specs=[
            pl.BlockSpec((B, 3), lambda: (0, 0)),
            pl.BlockSpec((1, 1), lambda: (0, 0)),
            plane, plane, plane, plane, plane,
        ],
        out_specs=pl.BlockSpec((B, _TOPN, 8), lambda: (0, 0, 0)),
        out_shape=jax.ShapeDtypeStruct((B, _TOPN, 8), jnp.float32),
        scratch_shapes=[
            pltpu.VMEM((_B, _TOPN, _LANES), jnp.float32),
            pltpu.VMEM((_B, _TOPN, _LANES), jnp.float32),
            pltpu.VMEM((_B, _TOPN, _LANES), jnp.float32),
            pltpu.VMEM((_B, _TOPN, _LANES), jnp.float32),
            pltpu.VMEM((_B, _TOPN, _LANES), jnp.float32),
            pltpu.VMEM((_B, _TOPN, _LANES), jnp.int32),
        ],
    )(im_info, td, sc, dxp, dyp, dwp, dhp)
    return out
